# Initial kernel scaffold; baseline (speedup 1.0000x reference)
import jax
import jax.numpy as jnp
from jax.experimental import pallas as pl


def kernel(x, edge_index, W1, b1, W2, b2):
    raise NotImplementedError("write your pallas kernel here")


# speedup vs baseline: 17.0822x; 17.0822x over previous
import functools

import jax
import jax.numpy as jnp
from jax import lax
from jax.experimental import pallas as pl
from jax.experimental.pallas import tpu as pltpu
from jax.experimental.pallas import tpu_sc as plsc

N_PAD = 10240
NC = 2
NS = 16
NW = NC * NS
CHUNK = 128
STRIPE = N_PAD // NS


def _sc_mesh():
    return plsc.VectorSubcoreMesh(core_axis_name="c", subcore_axis_name="s")


def _deg_body(dst_hbm, zeros1_hbm, out_hbm, idx_v, ones_v, deg_sh, sem):
    del sem
    cid = lax.axis_index("c")
    sid = lax.axis_index("s")
    gid = cid * NS + sid
    n_chunks = dst_hbm.shape[0] // CHUNK
    n_iter = (n_chunks + NW - 1) // NW

    stripe0 = pl.multiple_of(sid * STRIPE, 8)
    pltpu.sync_copy(zeros1_hbm.at[pl.ds(stripe0, STRIPE)],
                    deg_sh.at[pl.ds(stripe0, STRIPE)])
    for j in range(CHUNK // 16):
        ones_v[pl.ds(j * 16, 16)] = jnp.ones((16,), jnp.float32)
    plsc.subcore_barrier()

    def body(k, carry):
        chunk = k * NW + gid

        @pl.when(chunk < n_chunks)
        def _():
            off = pl.multiple_of(chunk * CHUNK, 8)
            pltpu.sync_copy(dst_hbm.at[pl.ds(off, CHUNK)], idx_v)
            pltpu.sync_copy(ones_v, deg_sh.at[idx_v], add=True)

        return carry

    lax.fori_loop(0, n_iter, body, 0)
    plsc.subcore_barrier()
    out0 = pl.multiple_of(cid * N_PAD + sid * STRIPE, 8)
    pltpu.sync_copy(deg_sh.at[pl.ds(stripe0, STRIPE)],
                    out_hbm.at[pl.ds(out0, STRIPE)])


def _make_deg_kernel(n_edges):
    del n_edges
    return pl.kernel(
        _deg_body,
        out_type=jax.ShapeDtypeStruct((NC * N_PAD,), jnp.float32),
        mesh=_sc_mesh(),
        scratch_types=[
            pltpu.VMEM((CHUNK,), jnp.int32),
            pltpu.VMEM((CHUNK,), jnp.float32),
            pltpu.VMEM_SHARED((N_PAD,), jnp.float32),
            pltpu.SemaphoreType.DMA,
        ],
    )


def _agg_body(g_hbm, src_hbm, dst_hbm, zeros2_hbm, out_hbm,
              src_v, dst_v, rows_v, agg_sh, sem):
    cid = lax.axis_index("c")
    sid = lax.axis_index("s")
    gid = cid * NS + sid
    n_chunks = src_hbm.shape[0] // CHUNK
    n_iter = (n_chunks + NW - 1) // NW

    stripe0 = pl.multiple_of(sid * STRIPE, 8)
    pltpu.sync_copy(zeros2_hbm.at[pl.ds(stripe0, STRIPE)],
                    agg_sh.at[pl.ds(stripe0, STRIPE)])
    plsc.subcore_barrier()

    def body(k, carry):
        chunk = k * NW + gid

        @pl.when(chunk < n_chunks)
        def _():
            off = pl.multiple_of(chunk * CHUNK, 8)
            pltpu.sync_copy(src_hbm.at[pl.ds(off, CHUNK)], src_v)
            pltpu.sync_copy(dst_hbm.at[pl.ds(off, CHUNK)], dst_v)
            pltpu.async_copy(g_hbm.at[src_v], rows_v, sem).wait()
            pltpu.sync_copy(rows_v, agg_sh.at[dst_v], add=True)

        return carry

    lax.fori_loop(0, n_iter, body, 0)
    plsc.subcore_barrier()
    out0 = pl.multiple_of(cid * N_PAD + sid * STRIPE, 8)
    pltpu.sync_copy(agg_sh.at[pl.ds(stripe0, STRIPE)],
                    out_hbm.at[pl.ds(out0, STRIPE)])


def _make_agg_kernel(d_model):
    return pl.kernel(
        _agg_body,
        out_type=jax.ShapeDtypeStruct((NC * N_PAD, d_model), jnp.float32),
        mesh=_sc_mesh(),
        compiler_params=pltpu.CompilerParams(use_tc_tiling_on_sc=False),
        scratch_types=[
            pltpu.VMEM((CHUNK,), jnp.int32),
            pltpu.VMEM((CHUNK,), jnp.int32),
            pltpu.VMEM((CHUNK, d_model), jnp.float32),
            pltpu.VMEM_SHARED((N_PAD, d_model), jnp.float32),
            pltpu.SemaphoreType.DMA,
        ],
    )


def _lin1_body(x_ref, w_ref, deg_ref, g_ref, dis_ref):
    deg = deg_ref[0, :] + deg_ref[1, :] + 1.0
    dis = jnp.where(deg > 0, lax.rsqrt(deg), 0.0)
    h = jnp.dot(x_ref[...], w_ref[...], preferred_element_type=jnp.float32)
    g_ref[...] = h * dis[:, None]
    dis_ref[...] = dis[:, None]


def _lin2_body(agg_ref, g1_ref, dis_ref, w_ref, b_ref, g2_ref):
    dis = dis_ref[...]
    agg = agg_ref[0] + agg_ref[1] + g1_ref[...]
    z = jnp.maximum(agg * dis + b_ref[...], 0.0)
    h2 = jnp.dot(z, w_ref[...], preferred_element_type=jnp.float32)
    g2_ref[...] = h2 * dis


def _out_body(agg_ref, g2_ref, dis_ref, b_ref, o_ref):
    agg = agg_ref[0] + agg_ref[1] + g2_ref[...]
    o_ref[...] = agg * dis_ref[...] + b_ref[...]


def _lin1(xp, w1, deg2):
    d_in, d_hid = w1.shape
    grid = (N_PAD // STRIPE,)
    return pl.pallas_call(
        _lin1_body,
        grid=grid,
        in_specs=[
            pl.BlockSpec((STRIPE, d_in), lambda i: (i, 0)),
            pl.BlockSpec((d_in, d_hid), lambda i: (0, 0)),
            pl.BlockSpec((NC, STRIPE), lambda i: (0, i)),
        ],
        out_specs=[
            pl.BlockSpec((STRIPE, d_hid), lambda i: (i, 0)),
            pl.BlockSpec((STRIPE, 1), lambda i: (i, 0)),
        ],
        out_shape=[
            jax.ShapeDtypeStruct((N_PAD, d_hid), jnp.float32),
            jax.ShapeDtypeStruct((N_PAD, 1), jnp.float32),
        ],
    )(xp, w1, deg2)


def _lin2(agg1, g1, dis, w2, b1):
    d_hid, d_out = w2.shape
    grid = (N_PAD // STRIPE,)
    return pl.pallas_call(
        _lin2_body,
        grid=grid,
        in_specs=[
            pl.BlockSpec((NC, STRIPE, d_hid), lambda i: (0, i, 0)),
            pl.BlockSpec((STRIPE, d_hid), lambda i: (i, 0)),
            pl.BlockSpec((STRIPE, 1), lambda i: (i, 0)),
            pl.BlockSpec((d_hid, d_out), lambda i: (0, 0)),
            pl.BlockSpec((1, d_hid), lambda i: (0, 0)),
        ],
        out_specs=pl.BlockSpec((STRIPE, d_out), lambda i: (i, 0)),
        out_shape=jax.ShapeDtypeStruct((N_PAD, d_out), jnp.float32),
    )(agg1, g1, dis, w2, b1)


def _outk(agg2, g2, dis, b2):
    d_out = g2.shape[1]
    grid = (N_PAD // STRIPE,)
    return pl.pallas_call(
        _out_body,
        grid=grid,
        in_specs=[
            pl.BlockSpec((NC, STRIPE, d_out), lambda i: (0, i, 0)),
            pl.BlockSpec((STRIPE, d_out), lambda i: (i, 0)),
            pl.BlockSpec((STRIPE, 1), lambda i: (i, 0)),
            pl.BlockSpec((1, d_out), lambda i: (0, 0)),
        ],
        out_specs=pl.BlockSpec((STRIPE, d_out), lambda i: (i, 0)),
        out_shape=jax.ShapeDtypeStruct((N_PAD, d_out), jnp.float32),
    )(agg2, g2, dis, b2)


@jax.jit
def kernel(x, edge_index, W1, b1, W2, b2):
    n, d_in = x.shape
    d_hid = W1.shape[1]
    d_out = W2.shape[1]
    src = edge_index[0].astype(jnp.int32)
    dst = edge_index[1].astype(jnp.int32)

    xp = jnp.zeros((N_PAD, d_in), jnp.float32).at[:n].set(x)
    zeros1 = jnp.zeros((N_PAD,), jnp.float32)
    zeros_h = jnp.zeros((N_PAD, d_hid), jnp.float32)
    zeros_o = jnp.zeros((N_PAD, d_out), jnp.float32)

    deg2 = _make_deg_kernel(dst.shape[0])(dst, zeros1).reshape(NC, N_PAD)
    g1, dis = _lin1(xp, W1, deg2)
    agg1 = _make_agg_kernel(d_hid)(g1, src, dst, zeros_h)
    agg1 = agg1.reshape(NC, N_PAD, d_hid)
    g2 = _lin2(agg1, g1, dis, W2, b1.reshape(1, d_hid))
    agg2 = _make_agg_kernel(d_out)(g2, src, dst, zeros_o)
    agg2 = agg2.reshape(NC, N_PAD, d_out)
    out = _outk(agg2, g2, dis, b2.reshape(1, d_out))
    return out[:n]
